# baseline (device time: 32512 ns/iter reference)
import jax
import jax.numpy as jnp
from jax import lax
from jax.experimental import pallas as pl
from jax.experimental.pallas import tpu as pltpu

N_DEV = 4
N_TOKENS = 512
D_MODEL = 256
D_OUT = 512
N_EXPERTS = 16
EXPERTS_PER_DEV = N_EXPERTS // N_DEV
CAPACITY = 25


def kernel(x, router_W, route_idx, expert_W):
    del router_W

    e_idx = route_idx[:, 0]
    onehot = e_idx[:, None] == jnp.arange(N_EXPERTS, dtype=jnp.int32)[None, :]
    pos = jnp.cumsum(onehot.astype(jnp.int32), axis=0)
    keep = jnp.logical_and(onehot, pos <= CAPACITY)
    my = lax.axis_index("i")
    local_mask = lax.dynamic_slice(
        keep.astype(jnp.float32),
        (0, my * EXPERTS_PER_DEV),
        (N_TOKENS, EXPERTS_PER_DEV),
    )

    def body(x_ref, w_ref, mask_ref, out_ref, comm_ref, send_sems, recv_sems):
        my_pos = lax.axis_index("i")
        left = (my_pos - 1) % N_DEV
        right = (my_pos + 1) % N_DEV

        barrier_sem = pltpu.get_barrier_semaphore()
        for nbr in [left, right]:
            pl.semaphore_signal(
                barrier_sem, inc=1,
                device_id=(nbr,), device_id_type=pl.DeviceIdType.MESH,
            )
        pl.semaphore_wait(barrier_sem, 2)

        x_bf = x_ref[:, :].astype(jnp.bfloat16)
        acc = jnp.zeros((N_TOKENS, D_OUT), jnp.float32)
        for le in range(EXPERTS_PER_DEV):
            xm = x_bf * mask_ref[:, le : le + 1].astype(jnp.bfloat16)
            acc += jnp.dot(
                xm,
                w_ref[le].astype(jnp.bfloat16),
                preferred_element_type=jnp.float32,
            )

        comm_ref[0] = acc.astype(jnp.bfloat16)
        total = acc

        for h in range(N_DEV - 1):
            send_slot = h % 2
            recv_slot = (h + 1) % 2
            rdma = pltpu.make_async_remote_copy(
                src_ref=comm_ref.at[send_slot],
                dst_ref=comm_ref.at[recv_slot],
                send_sem=send_sems.at[send_slot],
                recv_sem=recv_sems.at[recv_slot],
                device_id=(right,),
                device_id_type=pl.DeviceIdType.MESH,
            )
            rdma.start()
            rdma.wait()
            total += comm_ref[recv_slot].astype(jnp.float32)

        out_ref[:, :] = total.astype(jnp.bfloat16)

    return pl.pallas_call(
        body,
        out_shape=jax.ShapeDtypeStruct((N_TOKENS, D_OUT), jnp.bfloat16),
        in_specs=[
            pl.BlockSpec(memory_space=pltpu.VMEM),
            pl.BlockSpec(memory_space=pltpu.VMEM),
            pl.BlockSpec(memory_space=pltpu.VMEM),
        ],
        out_specs=pl.BlockSpec(memory_space=pltpu.VMEM),
        scratch_shapes=[
            pltpu.VMEM((2, N_TOKENS, D_OUT), jnp.bfloat16),
            pltpu.SemaphoreType.DMA((2,)),
            pltpu.SemaphoreType.DMA((2,)),
        ],
        compiler_params=pltpu.CompilerParams(collective_id=0),
    )(x, expert_W, local_mask)


# device time: 17869 ns/iter; 1.8195x vs baseline; 1.8195x over previous
import jax
import jax.numpy as jnp
from jax import lax
from jax.experimental import pallas as pl
from jax.experimental.pallas import tpu as pltpu

N_DEV = 4
N_TOKENS = 512
D_MODEL = 256
D_OUT = 512
N_EXPERTS = 16
EXPERTS_PER_DEV = N_EXPERTS // N_DEV
CAPACITY = 25
SLOTS = 32
ROWS_PER_DEV = EXPERTS_PER_DEV * SLOTS
TOTAL_ROWS = N_DEV * ROWS_PER_DEV


def kernel(x, router_W, route_idx, expert_W):
    del router_W

    e_idx = route_idx[:, 0]
    onehot = e_idx[:, None] == jnp.arange(N_EXPERTS, dtype=jnp.int32)[None, :]
    pos = jnp.cumsum(onehot.astype(jnp.int32), axis=0)
    pos_tok = jnp.take_along_axis(pos, e_idx[:, None], axis=1)[:, 0]
    kept = pos_tok <= CAPACITY
    row_id = jnp.where(kept, e_idx * SLOTS + pos_tok - 1, -1)

    P = (row_id[:, None] == jnp.arange(TOTAL_ROWS, dtype=jnp.int32)[None, :]).astype(
        jnp.bfloat16
    )
    my = lax.axis_index("i")
    local_rows = my * ROWS_PER_DEV + jnp.arange(ROWS_PER_DEV, dtype=jnp.int32)
    G = (row_id[None, :] == local_rows[:, None]).astype(jnp.bfloat16)

    def body(x_ref, w_ref, g_ref, p_ref, out_ref, comm_ref, send_sems, recv_sems):
        my_pos = lax.axis_index("i")

        barrier_sem = pltpu.get_barrier_semaphore()
        for j in range(1, N_DEV):
            pl.semaphore_signal(
                barrier_sem, inc=1,
                device_id=((my_pos + j) % N_DEV,),
                device_id_type=pl.DeviceIdType.MESH,
            )
        pl.semaphore_wait(barrier_sem, N_DEV - 1)

        x_bf = x_ref[:, :].astype(jnp.bfloat16)
        xg = jnp.dot(
            g_ref[:, :], x_bf, preferred_element_type=jnp.float32
        ).astype(jnp.bfloat16)
        blocks = [
            jnp.dot(
                xg[le * SLOTS : (le + 1) * SLOTS],
                w_ref[le].astype(jnp.bfloat16),
                preferred_element_type=jnp.float32,
            ).astype(jnp.bfloat16)
            for le in range(EXPERTS_PER_DEV)
        ]
        base = my_pos * ROWS_PER_DEV
        comm_ref[pl.ds(base, ROWS_PER_DEV), :] = jnp.concatenate(blocks, axis=0)

        rdmas = []
        for j in range(N_DEV - 1):
            rdma = pltpu.make_async_remote_copy(
                src_ref=comm_ref.at[pl.ds(base, ROWS_PER_DEV)],
                dst_ref=comm_ref.at[pl.ds(base, ROWS_PER_DEV)],
                send_sem=send_sems.at[j],
                recv_sem=recv_sems.at[j],
                device_id=((my_pos + 1 + j) % N_DEV,),
                device_id_type=pl.DeviceIdType.MESH,
            )
            rdma.start()
            rdmas.append(rdma)
        for rdma in rdmas:
            rdma.wait_recv()

        out_ref[:, :] = jnp.dot(
            p_ref[:, :], comm_ref[:, :], preferred_element_type=jnp.float32
        ).astype(jnp.bfloat16)

        for rdma in rdmas:
            rdma.wait_send()

    return pl.pallas_call(
        body,
        out_shape=jax.ShapeDtypeStruct((N_TOKENS, D_OUT), jnp.bfloat16),
        in_specs=[
            pl.BlockSpec(memory_space=pltpu.VMEM),
            pl.BlockSpec(memory_space=pltpu.VMEM),
            pl.BlockSpec(memory_space=pltpu.VMEM),
            pl.BlockSpec(memory_space=pltpu.VMEM),
        ],
        out_specs=pl.BlockSpec(memory_space=pltpu.VMEM),
        scratch_shapes=[
            pltpu.VMEM((TOTAL_ROWS, D_OUT), jnp.bfloat16),
            pltpu.SemaphoreType.DMA((N_DEV - 1,)),
            pltpu.SemaphoreType.DMA((N_DEV - 1,)),
        ],
        compiler_params=pltpu.CompilerParams(collective_id=0),
    )(x, expert_W, G, P)


# device time: 13784 ns/iter; 2.3587x vs baseline; 1.2964x over previous
import jax
import jax.numpy as jnp
from jax import lax
from jax.experimental import pallas as pl
from jax.experimental.pallas import tpu as pltpu

N_DEV = 4
N_TOKENS = 512
D_MODEL = 256
D_OUT = 512
N_EXPERTS = 16
EXPERTS_PER_DEV = N_EXPERTS // N_DEV
CAPACITY = 25
SLOTS = 32
ROWS_PER_DEV = EXPERTS_PER_DEV * SLOTS
TOTAL_ROWS = N_DEV * ROWS_PER_DEV


def kernel(x, router_W, route_idx, expert_W):
    del router_W

    def body(x_ref, idx_ref, w_ref, out_ref, comm_ref, send_sems, recv_sems):
        my_pos = lax.axis_index("i")

        barrier_sem = pltpu.get_barrier_semaphore()
        for j in range(1, N_DEV):
            pl.semaphore_signal(
                barrier_sem, inc=1,
                device_id=((my_pos + j) % N_DEV,),
                device_id_type=pl.DeviceIdType.MESH,
            )
        pl.semaphore_wait(barrier_sem, N_DEV - 1)

        toks = idx_ref[:, :]
        onehot = (
            toks == lax.broadcasted_iota(jnp.int32, (N_TOKENS, N_EXPERTS), 1)
        ).astype(jnp.bfloat16)
        lt = (
            lax.broadcasted_iota(jnp.int32, (N_TOKENS, N_TOKENS), 1)
            <= lax.broadcasted_iota(jnp.int32, (N_TOKENS, N_TOKENS), 0)
        ).astype(jnp.bfloat16)
        pos = jnp.dot(lt, onehot, preferred_element_type=jnp.float32)
        pos_tok = jnp.sum(
            pos * onehot.astype(jnp.float32), axis=1, keepdims=True
        ).astype(jnp.int32)
        kept = pos_tok <= CAPACITY
        row_id = jnp.where(kept, toks * SLOTS + pos_tok - 1, -1)

        p_mat = (
            row_id == lax.broadcasted_iota(jnp.int32, (N_TOKENS, TOTAL_ROWS), 1)
        ).astype(jnp.bfloat16)

        base = my_pos * ROWS_PER_DEV
        g_t = (
            row_id
            == base + lax.broadcasted_iota(jnp.int32, (N_TOKENS, ROWS_PER_DEV), 1)
        ).astype(jnp.bfloat16)
        x_bf = x_ref[:, :].astype(jnp.bfloat16)
        xg = lax.dot_general(
            g_t, x_bf,
            dimension_numbers=(((0,), (0,)), ((), ())),
            preferred_element_type=jnp.float32,
        ).astype(jnp.bfloat16)

        blocks = [
            jnp.dot(
                xg[le * SLOTS : (le + 1) * SLOTS],
                w_ref[le].astype(jnp.bfloat16),
                preferred_element_type=jnp.float32,
            ).astype(jnp.bfloat16)
            for le in range(EXPERTS_PER_DEV)
        ]
        comm_ref[pl.ds(base, ROWS_PER_DEV), :] = jnp.concatenate(blocks, axis=0)

        rdmas = []
        for j in range(N_DEV - 1):
            rdma = pltpu.make_async_remote_copy(
                src_ref=comm_ref.at[pl.ds(base, ROWS_PER_DEV)],
                dst_ref=comm_ref.at[pl.ds(base, ROWS_PER_DEV)],
                send_sem=send_sems.at[j],
                recv_sem=recv_sems.at[j],
                device_id=((my_pos + 1 + j) % N_DEV,),
                device_id_type=pl.DeviceIdType.MESH,
            )
            rdma.start()
            rdmas.append(rdma)
        for rdma in rdmas:
            rdma.wait_recv()

        out_ref[:, :] = jnp.dot(
            p_mat, comm_ref[:, :], preferred_element_type=jnp.float32
        ).astype(jnp.bfloat16)

        for rdma in rdmas:
            rdma.wait_send()

    return pl.pallas_call(
        body,
        out_shape=jax.ShapeDtypeStruct((N_TOKENS, D_OUT), jnp.bfloat16),
        in_specs=[
            pl.BlockSpec(memory_space=pltpu.VMEM),
            pl.BlockSpec(memory_space=pltpu.VMEM),
            pl.BlockSpec(memory_space=pltpu.VMEM),
        ],
        out_specs=pl.BlockSpec(memory_space=pltpu.VMEM),
        scratch_shapes=[
            pltpu.VMEM((TOTAL_ROWS, D_OUT), jnp.bfloat16),
            pltpu.SemaphoreType.DMA((N_DEV - 1,)),
            pltpu.SemaphoreType.DMA((N_DEV - 1,)),
        ],
        compiler_params=pltpu.CompilerParams(collective_id=0),
    )(x, route_idx, expert_W)


# device time: 12426 ns/iter; 2.6164x vs baseline; 1.1093x over previous
import jax
import jax.numpy as jnp
from jax import lax
from jax.experimental import pallas as pl
from jax.experimental.pallas import tpu as pltpu

N_DEV = 4
N_TOKENS = 512
D_MODEL = 256
D_OUT = 512
N_EXPERTS = 16
EXPERTS_PER_DEV = N_EXPERTS // N_DEV
CAPACITY = 25
SLOTS = 32
ROWS_PER_DEV = EXPERTS_PER_DEV * SLOTS
TOTAL_ROWS = N_DEV * ROWS_PER_DEV


def kernel(x, router_W, route_idx, expert_W):
    del router_W

    def body(x_ref, idx_ref, w_ref, out_ref, comm_ref, send_sems, recv_sems):
        my_pos = lax.axis_index("i")

        barrier_sem = pltpu.get_barrier_semaphore()
        for j in range(1, N_DEV):
            pl.semaphore_signal(
                barrier_sem, inc=1,
                device_id=((my_pos + j) % N_DEV,),
                device_id_type=pl.DeviceIdType.MESH,
            )

        toks = idx_ref[:, :]
        onehot = (
            toks == lax.broadcasted_iota(jnp.int32, (N_TOKENS, N_EXPERTS), 1)
        ).astype(jnp.bfloat16)
        lt = (
            lax.broadcasted_iota(jnp.int32, (N_TOKENS, N_TOKENS), 1)
            <= lax.broadcasted_iota(jnp.int32, (N_TOKENS, N_TOKENS), 0)
        ).astype(jnp.bfloat16)
        pos = jnp.dot(lt, onehot, preferred_element_type=jnp.float32)
        pos_tok = jnp.sum(
            pos * onehot.astype(jnp.float32), axis=1, keepdims=True
        ).astype(jnp.int32)
        kept = pos_tok <= CAPACITY
        row_id = jnp.where(kept, toks * SLOTS + pos_tok - 1, -1)

        base = my_pos * ROWS_PER_DEV
        g_t = (
            row_id
            == base + lax.broadcasted_iota(jnp.int32, (N_TOKENS, ROWS_PER_DEV), 1)
        ).astype(jnp.bfloat16)
        x_bf = x_ref[:, :].astype(jnp.bfloat16)
        xg = lax.dot_general(
            g_t, x_bf,
            dimension_numbers=(((0,), (0,)), ((), ())),
            preferred_element_type=jnp.float32,
        ).astype(jnp.bfloat16)

        blocks = [
            jnp.dot(
                xg[le * SLOTS : (le + 1) * SLOTS],
                w_ref[le].astype(jnp.bfloat16),
                preferred_element_type=jnp.float32,
            ).astype(jnp.bfloat16)
            for le in range(EXPERTS_PER_DEV)
        ]
        comm_ref[pl.ds(base, ROWS_PER_DEV), :] = jnp.concatenate(blocks, axis=0)

        pl.semaphore_wait(barrier_sem, N_DEV - 1)
        rdmas = []
        for j in range(N_DEV - 1):
            rdma = pltpu.make_async_remote_copy(
                src_ref=comm_ref.at[pl.ds(base, ROWS_PER_DEV)],
                dst_ref=comm_ref.at[pl.ds(base, ROWS_PER_DEV)],
                send_sem=send_sems.at[j],
                recv_sem=recv_sems.at[j],
                device_id=((my_pos + 1 + j) % N_DEV,),
                device_id_type=pl.DeviceIdType.MESH,
            )
            rdma.start()
            rdmas.append(rdma)

        def scatter_block(src_dev):
            p_blk = (
                row_id
                == src_dev * ROWS_PER_DEV
                + lax.broadcasted_iota(jnp.int32, (N_TOKENS, ROWS_PER_DEV), 1)
            ).astype(jnp.bfloat16)
            return jnp.dot(
                p_blk,
                comm_ref[pl.ds(src_dev * ROWS_PER_DEV, ROWS_PER_DEV), :],
                preferred_element_type=jnp.float32,
            )

        out_acc = scatter_block(my_pos)
        for j, rdma in enumerate(rdmas):
            rdma.wait_recv()
            out_acc += scatter_block((my_pos + 1 + j) % N_DEV)
        out_ref[:, :] = out_acc.astype(jnp.bfloat16)

        for rdma in rdmas:
            rdma.wait_send()

    return pl.pallas_call(
        body,
        out_shape=jax.ShapeDtypeStruct((N_TOKENS, D_OUT), jnp.bfloat16),
        in_specs=[
            pl.BlockSpec(memory_space=pltpu.VMEM),
            pl.BlockSpec(memory_space=pltpu.VMEM),
            pl.BlockSpec(memory_space=pltpu.VMEM),
        ],
        out_specs=pl.BlockSpec(memory_space=pltpu.VMEM),
        scratch_shapes=[
            pltpu.VMEM((TOTAL_ROWS, D_OUT), jnp.bfloat16),
            pltpu.SemaphoreType.DMA((N_DEV - 1,)),
            pltpu.SemaphoreType.DMA((N_DEV - 1,)),
        ],
        compiler_params=pltpu.CompilerParams(collective_id=0),
    )(x, route_idx, expert_W)
